# Initial kernel scaffold; baseline (speedup 1.0000x reference)
import jax
import jax.numpy as jnp
from jax.experimental import pallas as pl


def kernel(h, edge_index, edge_relation, node_color_rep, node_role, rel_emb, role_emb, color_emb, We1, be1, We2, be2, Wn1, bn1, Wn2, bn2, ln_g, ln_b):
    raise NotImplementedError("write your pallas kernel here")


# speedup vs baseline: 17.6230x; 17.6230x over previous
import functools

import jax
import jax.numpy as jnp
from jax import lax
from jax.experimental import pallas as pl
from jax.experimental.pallas import tpu as pltpu
from jax.experimental.pallas import tpu_sc as plsc

N = 10000
E = 320000
D = 128

NC = 2
NS = 16
NW = NC * NS
CH = 128
CPW = 79
EPW = CPW * CH
E_PAD = NW * EPW
N_ACC = 10112
ZCH = 5

_mesh = lambda: plsc.VectorSubcoreMesh(core_axis_name="c", subcore_axis_name="s")


def _prep_body(h_ref, roh_ref, coh_ref, rel_ref, rolemb_ref, colemb_ref,
               we1_ref, be1_ref, wn1_ref, bn1_ref,
               src_ref, dst_ref, relb_ref, extra_ref):
    f32 = jnp.float32
    h = h_ref[...]
    roh = roh_ref[...]
    coh = coh_ref[...]
    rolemb = rolemb_ref[...]
    colemb = colemb_ref[...]
    dot = functools.partial(jnp.dot, preferred_element_type=f32)
    src = dot(h, we1_ref[0:128, :])
    src += dot(roh, dot(rolemb, we1_ref[272:280, :]))
    src += dot(coh, dot(colemb, we1_ref[288:296, :]))
    dst = dot(h, we1_ref[128:256, :])
    dst += dot(roh, dot(rolemb, we1_ref[280:288, :]))
    dst += dot(coh, dot(colemb, we1_ref[296:304, :]))
    src_ref[...] = src
    dst_ref[...] = dst
    relb_ref[...] = dot(rel_ref[...], we1_ref[256:272, :]) + be1_ref[...]
    extra = dot(roh, dot(rolemb, wn1_ref[256:264, :]))
    extra += dot(coh, dot(colemb, wn1_ref[264:272, :]))
    extra_ref[...] = extra + bn1_ref[...]


def _prep(h, roh, coh, rel_emb, rolemb, colemb, We1, be1, Wn1, bn1):
    return pl.pallas_call(
        _prep_body,
        out_shape=(
            jax.ShapeDtypeStruct((N, D), jnp.float32),
            jax.ShapeDtypeStruct((N, D), jnp.float32),
            jax.ShapeDtypeStruct((8, D), jnp.float32),
            jax.ShapeDtypeStruct((N, D), jnp.float32),
        ),
    )(h, roh, coh, rel_emb, rolemb, colemb, We1, be1.reshape(1, D), Wn1,
      bn1.reshape(1, D))


def _gather_body(src_hbm, dst_hbm, srci_hbm, dsti_hbm, out_hbm,
                 idxs, idxd, bufs, bufd, bufo,
                 gs0, gs1, os0, os1):
    wid = lax.axis_index("s") * NC + lax.axis_index("c")
    base = wid * EPW
    pltpu.sync_copy(srci_hbm.at[wid], idxs)
    pltpu.sync_copy(dsti_hbm.at[wid], idxd)
    gsems = (gs0, gs1)
    osems = (os0, os1)

    def issue(j, b):
        pltpu.async_copy(src_hbm.at[idxs.at[j]], bufs.at[b], gsems[b])
        pltpu.async_copy(dst_hbm.at[idxd.at[j]], bufd.at[b], gsems[b])

    def wait_gather(j, b):
        pltpu.make_async_copy(src_hbm.at[idxs.at[j]], bufs.at[b], gsems[b]).wait()
        pltpu.make_async_copy(dst_hbm.at[idxd.at[j]], bufd.at[b], gsems[b]).wait()

    def add_rows(b):
        def row(r, _):
            for c in range(D // 16):
                s = pl.ds(c * 16, 16)
                bufo[b, r, s] = bufs[b, r, s] + bufd[b, r, s]
            return 0
        lax.fori_loop(0, CH, row, 0)

    def drain_out(j, b):
        pltpu.make_async_copy(
            bufo.at[b], out_hbm.at[pl.ds(base + j * CH, CH), :], osems[b]).wait()

    issue(0, 0)
    issue(1, 1)

    def outer(i, _):
        j0 = i * 2
        for b in range(2):
            j = j0 + b
            wait_gather(j, b)

            @pl.when(j >= 2)
            def _():
                drain_out(j, b)

            add_rows(b)

            @pl.when(j + 2 < CPW)
            def _():
                issue(j + 2, b)

            pltpu.async_copy(
                bufo.at[b], out_hbm.at[pl.ds(base + j * CH, CH), :], osems[b])
        return 0

    lax.fori_loop(0, (CPW - 1) // 2, outer, 0)
    j = CPW - 1
    wait_gather(j, 0)
    drain_out(j, 0)
    add_rows(0)
    pltpu.async_copy(bufo.at[0], out_hbm.at[pl.ds(base + j * CH, CH), :], os0)
    drain_out(j, 0)
    pltpu.make_async_copy(
        bufo.at[1], out_hbm.at[pl.ds(base, CH), :], os1).wait()


def _gather(src_tab, dst_tab, srci, dsti):
    k = pl.kernel(
        _gather_body,
        out_type=jax.ShapeDtypeStruct((E_PAD, D), jnp.float32),
        mesh=_mesh(),
        scratch_types=[
            pltpu.VMEM((CPW, CH), jnp.int32),
            pltpu.VMEM((CPW, CH), jnp.int32),
            pltpu.VMEM((2, CH, D), jnp.float32),
            pltpu.VMEM((2, CH, D), jnp.float32),
            pltpu.VMEM((2, CH, D), jnp.float32),
            pltpu.SemaphoreType.DMA,
            pltpu.SemaphoreType.DMA,
            pltpu.SemaphoreType.DMA,
            pltpu.SemaphoreType.DMA,
        ],
    )
    return k(src_tab, dst_tab, srci, dsti)


BE = 4096


def _edge_body(pre_ref, oh_ref, relb_ref, we2_ref, be2_ref, out_ref):
    f32 = jnp.float32
    pre1 = pre_ref[...] + jnp.dot(oh_ref[...], relb_ref[...],
                                  preferred_element_type=f32)
    hid = pre1 * jax.nn.sigmoid(pre1)
    m = jnp.dot(hid, we2_ref[...], preferred_element_type=f32) + be2_ref[0:1, :]
    out_ref[...] = m * jax.nn.sigmoid(m)


def _edge_mlp(pre, oh, relb, We2, be2):
    return pl.pallas_call(
        _edge_body,
        grid=(E_PAD // BE,),
        in_specs=[
            pl.BlockSpec((BE, D), lambda i: (i, 0)),
            pl.BlockSpec((BE, 8), lambda i: (i, 0)),
            pl.BlockSpec((8, D), lambda i: (0, 0)),
            pl.BlockSpec((D, D), lambda i: (0, 0)),
            pl.BlockSpec((8, D), lambda i: (0, 0)),
        ],
        out_specs=pl.BlockSpec((BE, D), lambda i: (i, 0)),
        out_shape=jax.ShapeDtypeStruct((E_PAD, D), jnp.float32),
    )(pre, oh, relb, We2, jnp.broadcast_to(be2.reshape(1, D), (8, D)))


def _scatter(m, dsti, zeros128):
    def body(m_hbm, dsti_hbm, zeros_hbm, out_hbm,
             agg_sp, idxd, bufm, s0, s1):
        core = lax.axis_index("c")
        sid = lax.axis_index("s")
        wid = sid * NC + core
        base = wid * EPW
        sems = (s0, s1)

        pltpu.sync_copy(zeros_hbm, bufm.at[0])
        for k in range(ZCH):
            cid = sid * ZCH + k

            @pl.when(cid < CPW)
            def _():
                pltpu.sync_copy(bufm.at[0], agg_sp.at[pl.ds(cid * CH, CH), :])

        pltpu.sync_copy(dsti_hbm.at[wid], idxd)
        plsc.subcore_barrier()

        def issue(j, b):
            pltpu.async_copy(
                m_hbm.at[pl.ds(base + j * CH, CH), :], bufm.at[b], sems[b])

        def wait_in(j, b):
            pltpu.make_async_copy(
                m_hbm.at[pl.ds(base + j * CH, CH), :], bufm.at[b], sems[b]).wait()

        issue(0, 0)
        issue(1, 1)

        def outer(i, _):
            j0 = i * 2
            for b in range(2):
                j = j0 + b
                wait_in(j, b)
                pltpu.sync_copy(bufm.at[b], agg_sp.at[idxd.at[j]], add=True)

                @pl.when(j + 2 < CPW)
                def _():
                    issue(j + 2, b)
            return 0

        lax.fori_loop(0, (CPW - 1) // 2, outer, 0)
        j = CPW - 1
        wait_in(j, 0)
        pltpu.sync_copy(bufm.at[0], agg_sp.at[idxd.at[j]], add=True)

        plsc.subcore_barrier()
        for k in range(ZCH):
            cid = sid * ZCH + k

            @pl.when(cid < CPW)
            def _():
                pltpu.sync_copy(agg_sp.at[pl.ds(cid * CH, CH), :], bufm.at[0])
                pltpu.sync_copy(bufm.at[0],
                                out_hbm.at[core, pl.ds(cid * CH, CH), :])

    k = pl.kernel(
        body,
        out_type=jax.ShapeDtypeStruct((NC, N_ACC, D), jnp.float32),
        mesh=_mesh(),
        scratch_types=[
            pltpu.VMEM_SHARED((N_ACC, D), jnp.float32),
            pltpu.VMEM((CPW, CH), jnp.int32),
            pltpu.VMEM((2, CH, D), jnp.float32),
            pltpu.SemaphoreType.DMA,
            pltpu.SemaphoreType.DMA,
        ],
    )
    return k(m, dsti, zeros128)


def _node_body(h_ref, agg_ref, extra_ref, wn1_ref, wn2_ref, bn2_ref,
               g_ref, b_ref, out_ref):
    f32 = jnp.float32
    h = h_ref[...]
    agg = agg_ref[0, :N, :] + agg_ref[1, :N, :]
    pre = jnp.dot(h, wn1_ref[0:128, :], preferred_element_type=f32)
    pre += jnp.dot(agg, wn1_ref[128:256, :], preferred_element_type=f32)
    pre += extra_ref[...]
    u = pre * jax.nn.sigmoid(pre)
    upd = jnp.dot(u, wn2_ref[...], preferred_element_type=f32) + bn2_ref[...]
    x = h + upd
    mu = jnp.mean(x, axis=-1, keepdims=True)
    xc = x - mu
    var = jnp.mean(xc * xc, axis=-1, keepdims=True)
    out_ref[...] = xc * lax.rsqrt(var + 1e-5) * g_ref[...] + b_ref[...]


def _node_mlp(h, aggpair, extra, Wn1, Wn2, bn2, g, b):
    return pl.pallas_call(
        _node_body,
        out_shape=jax.ShapeDtypeStruct((N, D), jnp.float32),
    )(h, aggpair, extra, Wn1, Wn2, bn2.reshape(1, D), g.reshape(1, D),
      b.reshape(1, D))


def kernel(h, edge_index, edge_relation, node_color_rep, node_role,
           rel_emb, role_emb, color_emb,
           We1, be1, We2, be2, Wn1, bn1, Wn2, bn2, ln_g, ln_b):
    i32 = jnp.int32
    f32 = jnp.float32
    src = edge_index[0].astype(i32)
    dst = edge_index[1].astype(i32)
    rel = edge_relation.astype(i32)
    pad = E_PAD - E
    srci = jnp.concatenate([src, jnp.zeros((pad,), i32)]).reshape(NW, CPW, CH)
    dsti_g = jnp.concatenate([dst, jnp.zeros((pad,), i32)]).reshape(NW, CPW, CH)
    dsti_s = jnp.concatenate([dst, jnp.full((pad,), N, i32)]).reshape(NW, CPW, CH)
    rel_p = jnp.concatenate([rel, jnp.zeros((pad,), i32)])
    rel_oh = (rel_p[:, None] == jnp.arange(8, dtype=i32)[None, :]).astype(f32)
    role_oh = (node_role.astype(i32)[:, None]
               == jnp.arange(8, dtype=i32)[None, :]).astype(f32)
    color_oh = (node_color_rep.astype(i32)[:, None]
                == jnp.arange(8, dtype=i32)[None, :]).astype(f32)
    rolemb = jnp.zeros((8, 8), f32).at[:4, :].set(role_emb.astype(f32))
    colemb = jnp.zeros((8, 8), f32).at[:3, :].set(color_emb.astype(f32))

    src_tab, dst_tab, relb, extra = _prep(
        h, role_oh, color_oh, rel_emb, rolemb, colemb, We1, be1, Wn1, bn1)
    pre_partial = _gather(src_tab, dst_tab, srci, dsti_g)
    m = _edge_mlp(pre_partial, rel_oh, relb, We2, be2)
    aggpair = _scatter(m, dsti_s, jnp.zeros((CH, D), f32))
    return _node_mlp(h, aggpair, extra, Wn1, Wn2, bn2, ln_g, ln_b)


# speedup vs baseline: 20.5747x; 1.1675x over previous
import functools

import jax
import jax.numpy as jnp
from jax import lax
from jax.experimental import pallas as pl
from jax.experimental.pallas import tpu as pltpu
from jax.experimental.pallas import tpu_sc as plsc

N = 10000
E = 320000
D = 128

NC = 2
NS = 16
NW = NC * NS
CH = 128
CPW = 79
EPW = CPW * CH
E_PAD = NW * EPW
N_ACC = 10112
ZCH = 5

_mesh = lambda: plsc.VectorSubcoreMesh(core_axis_name="c", subcore_axis_name="s")


def _prep_body(h_ref, roh_ref, coh_ref, rel_ref, rolemb_ref, colemb_ref,
               we1_ref, be1_ref, wn1_ref, bn1_ref,
               src_ref, dst_ref, relb_ref, extra_ref):
    f32 = jnp.float32
    h = h_ref[...]
    iota8 = lax.broadcasted_iota(jnp.int32, (1, 8), 1)
    roh = (roh_ref[...] == iota8).astype(f32)
    coh = (coh_ref[...] == iota8).astype(f32)
    rolemb = rolemb_ref[...]
    colemb = colemb_ref[...]
    dot = functools.partial(jnp.dot, preferred_element_type=f32)
    src = dot(h, we1_ref[0:128, :])
    src += dot(roh, dot(rolemb, we1_ref[272:280, :]))
    src += dot(coh, dot(colemb, we1_ref[288:296, :]))
    dst = dot(h, we1_ref[128:256, :])
    dst += dot(roh, dot(rolemb, we1_ref[280:288, :]))
    dst += dot(coh, dot(colemb, we1_ref[296:304, :]))
    src_ref[...] = src
    dst_ref[...] = dst
    relb_ref[...] = dot(rel_ref[...], we1_ref[256:272, :]) + be1_ref[...]
    extra = dot(roh, dot(rolemb, wn1_ref[256:264, :]))
    extra += dot(coh, dot(colemb, wn1_ref[264:272, :]))
    extra_ref[...] = extra + bn1_ref[...]


def _prep(h, roh, coh, rel_emb, rolemb, colemb, We1, be1, Wn1, bn1):
    return pl.pallas_call(
        _prep_body,
        out_shape=(
            jax.ShapeDtypeStruct((N, D), jnp.float32),
            jax.ShapeDtypeStruct((N, D), jnp.float32),
            jax.ShapeDtypeStruct((8, D), jnp.float32),
            jax.ShapeDtypeStruct((N, D), jnp.float32),
        ),
    )(h, roh, coh, rel_emb, rolemb, colemb, We1, be1.reshape(1, D), Wn1,
      bn1.reshape(1, D))


def _gather_body(src_hbm, dst_hbm, srci_hbm, dsti_hbm, out_hbm,
                 idxs, idxd, bufs, bufd, bufo,
                 gs0, gs1, os0, os1):
    wid = lax.axis_index("s") * NC + lax.axis_index("c")
    base = wid * EPW
    pltpu.sync_copy(srci_hbm.at[wid], idxs)
    pltpu.sync_copy(dsti_hbm.at[wid], idxd)
    gsems = (gs0, gs1)
    osems = (os0, os1)

    def issue(j, b):
        pltpu.async_copy(src_hbm.at[idxs.at[j]], bufs.at[b], gsems[b])
        pltpu.async_copy(dst_hbm.at[idxd.at[j]], bufd.at[b], gsems[b])

    def wait_gather(j, b):
        pltpu.make_async_copy(src_hbm.at[idxs.at[j]], bufs.at[b], gsems[b]).wait()
        pltpu.make_async_copy(dst_hbm.at[idxd.at[j]], bufd.at[b], gsems[b]).wait()

    def add_rows(b):
        def row(r, _):
            for c in range(D // 16):
                s = pl.ds(c * 16, 16)
                bufo[b, r, s] = bufs[b, r, s] + bufd[b, r, s]
            return 0
        lax.fori_loop(0, CH, row, 0)

    def drain_out(j, b):
        pltpu.make_async_copy(
            bufo.at[b], out_hbm.at[pl.ds(base + j * CH, CH), :], osems[b]).wait()

    issue(0, 0)
    issue(1, 1)

    def outer(i, _):
        j0 = i * 2
        for b in range(2):
            j = j0 + b
            wait_gather(j, b)

            @pl.when(j >= 2)
            def _():
                drain_out(j, b)

            add_rows(b)

            @pl.when(j + 2 < CPW)
            def _():
                issue(j + 2, b)

            pltpu.async_copy(
                bufo.at[b], out_hbm.at[pl.ds(base + j * CH, CH), :], osems[b])
        return 0

    lax.fori_loop(0, (CPW - 1) // 2, outer, 0)
    j = CPW - 1
    wait_gather(j, 0)
    drain_out(j, 0)
    add_rows(0)
    pltpu.async_copy(bufo.at[0], out_hbm.at[pl.ds(base + j * CH, CH), :], os0)
    drain_out(j, 0)
    pltpu.make_async_copy(
        bufo.at[1], out_hbm.at[pl.ds(base, CH), :], os1).wait()


def _gather(src_tab, dst_tab, srci, dsti):
    k = pl.kernel(
        _gather_body,
        out_type=jax.ShapeDtypeStruct((E_PAD, D), jnp.float32),
        mesh=_mesh(),
        scratch_types=[
            pltpu.VMEM((CPW, CH), jnp.int32),
            pltpu.VMEM((CPW, CH), jnp.int32),
            pltpu.VMEM((2, CH, D), jnp.float32),
            pltpu.VMEM((2, CH, D), jnp.float32),
            pltpu.VMEM((2, CH, D), jnp.float32),
            pltpu.SemaphoreType.DMA,
            pltpu.SemaphoreType.DMA,
            pltpu.SemaphoreType.DMA,
            pltpu.SemaphoreType.DMA,
        ],
    )
    return k(src_tab, dst_tab, srci, dsti)


BE = 4096


def _edge_body(pre_ref, rel_ref, relb_ref, we2_ref, be2_ref, out_ref):
    f32 = jnp.float32
    rel = rel_ref[0, 0, :]
    oh = (rel[:, None] == lax.broadcasted_iota(jnp.int32, (1, 8), 1)).astype(f32)
    pre1 = pre_ref[...] + jnp.dot(oh, relb_ref[...],
                                  preferred_element_type=f32)
    hid = pre1 * jax.nn.sigmoid(pre1)
    m = jnp.dot(hid, we2_ref[...], preferred_element_type=f32) + be2_ref[0:1, :]
    out_ref[...] = m * jax.nn.sigmoid(m)


def _edge_mlp(pre, rel3d, relb, We2, be2):
    return pl.pallas_call(
        _edge_body,
        grid=(E_PAD // BE,),
        in_specs=[
            pl.BlockSpec((BE, D), lambda i: (i, 0)),
            pl.BlockSpec((1, 1, BE), lambda i: (i, 0, 0)),
            pl.BlockSpec((8, D), lambda i: (0, 0)),
            pl.BlockSpec((D, D), lambda i: (0, 0)),
            pl.BlockSpec((8, D), lambda i: (0, 0)),
        ],
        out_specs=pl.BlockSpec((BE, D), lambda i: (i, 0)),
        out_shape=jax.ShapeDtypeStruct((E_PAD, D), jnp.float32),
    )(pre, rel3d, relb, We2, jnp.broadcast_to(be2.reshape(1, D), (8, D)))


def _scatter(m, dsti, zeros128):
    def body(m_hbm, dsti_hbm, zeros_hbm, out_hbm,
             agg_sp, idxd, bufm, s0, s1):
        core = lax.axis_index("c")
        sid = lax.axis_index("s")
        wid = sid * NC + core
        base = wid * EPW
        sems = (s0, s1)

        pltpu.sync_copy(zeros_hbm, bufm.at[0])
        for k in range(ZCH):
            cid = sid * ZCH + k

            @pl.when(cid < CPW)
            def _():
                pltpu.sync_copy(bufm.at[0], agg_sp.at[pl.ds(cid * CH, CH), :])

        pltpu.sync_copy(dsti_hbm.at[wid], idxd)
        plsc.subcore_barrier()

        def issue(j, b):
            pltpu.async_copy(
                m_hbm.at[pl.ds(base + j * CH, CH), :], bufm.at[b], sems[b])

        def wait_in(j, b):
            pltpu.make_async_copy(
                m_hbm.at[pl.ds(base + j * CH, CH), :], bufm.at[b], sems[b]).wait()

        issue(0, 0)
        issue(1, 1)

        def outer(i, _):
            j0 = i * 2
            for b in range(2):
                j = j0 + b
                wait_in(j, b)
                pltpu.sync_copy(bufm.at[b], agg_sp.at[idxd.at[j]], add=True)

                @pl.when(j + 2 < CPW)
                def _():
                    issue(j + 2, b)
            return 0

        lax.fori_loop(0, (CPW - 1) // 2, outer, 0)
        j = CPW - 1
        wait_in(j, 0)
        pltpu.sync_copy(bufm.at[0], agg_sp.at[idxd.at[j]], add=True)

        plsc.subcore_barrier()
        for k in range(ZCH):
            cid = sid * ZCH + k

            @pl.when(cid < CPW)
            def _():
                pltpu.sync_copy(agg_sp.at[pl.ds(cid * CH, CH), :], bufm.at[0])
                pltpu.sync_copy(bufm.at[0],
                                out_hbm.at[core, pl.ds(cid * CH, CH), :])

    k = pl.kernel(
        body,
        out_type=jax.ShapeDtypeStruct((NC, N_ACC, D), jnp.float32),
        mesh=_mesh(),
        scratch_types=[
            pltpu.VMEM_SHARED((N_ACC, D), jnp.float32),
            pltpu.VMEM((CPW, CH), jnp.int32),
            pltpu.VMEM((2, CH, D), jnp.float32),
            pltpu.SemaphoreType.DMA,
            pltpu.SemaphoreType.DMA,
        ],
    )
    return k(m, dsti, zeros128)


def _node_body(h_ref, agg_ref, extra_ref, wn1_ref, wn2_ref, bn2_ref,
               g_ref, b_ref, out_ref):
    f32 = jnp.float32
    h = h_ref[...]
    agg = agg_ref[0, :N, :] + agg_ref[1, :N, :]
    pre = jnp.dot(h, wn1_ref[0:128, :], preferred_element_type=f32)
    pre += jnp.dot(agg, wn1_ref[128:256, :], preferred_element_type=f32)
    pre += extra_ref[...]
    u = pre * jax.nn.sigmoid(pre)
    upd = jnp.dot(u, wn2_ref[...], preferred_element_type=f32) + bn2_ref[...]
    x = h + upd
    mu = jnp.mean(x, axis=-1, keepdims=True)
    xc = x - mu
    var = jnp.mean(xc * xc, axis=-1, keepdims=True)
    out_ref[...] = xc * lax.rsqrt(var + 1e-5) * g_ref[...] + b_ref[...]


def _node_mlp(h, aggpair, extra, Wn1, Wn2, bn2, g, b):
    return pl.pallas_call(
        _node_body,
        out_shape=jax.ShapeDtypeStruct((N, D), jnp.float32),
    )(h, aggpair, extra, Wn1, Wn2, bn2.reshape(1, D), g.reshape(1, D),
      b.reshape(1, D))


def kernel(h, edge_index, edge_relation, node_color_rep, node_role,
           rel_emb, role_emb, color_emb,
           We1, be1, We2, be2, Wn1, bn1, Wn2, bn2, ln_g, ln_b):
    i32 = jnp.int32
    f32 = jnp.float32
    src = edge_index[0].astype(i32)
    dst = edge_index[1].astype(i32)
    rel = edge_relation.astype(i32)
    pad = E_PAD - E
    srci = jnp.concatenate([src, jnp.zeros((pad,), i32)]).reshape(NW, CPW, CH)
    dsti_g = jnp.concatenate([dst, jnp.zeros((pad,), i32)]).reshape(NW, CPW, CH)
    dsti_s = jnp.concatenate([dst, jnp.full((pad,), N, i32)]).reshape(NW, CPW, CH)
    rel3d = jnp.concatenate([rel, jnp.zeros((pad,), i32)]).reshape(
        E_PAD // BE, 1, BE)
    role_c = node_role.astype(i32).reshape(N, 1)
    color_c = node_color_rep.astype(i32).reshape(N, 1)
    rolemb = jnp.zeros((8, 8), f32).at[:4, :].set(role_emb.astype(f32))
    colemb = jnp.zeros((8, 8), f32).at[:3, :].set(color_emb.astype(f32))

    src_tab, dst_tab, relb, extra = _prep(
        h, role_c, color_c, rel_emb, rolemb, colemb, We1, be1, Wn1, bn1)
    pre_partial = _gather(src_tab, dst_tab, srci, dsti_g)
    m = _edge_mlp(pre_partial, rel3d, relb, We2, be2)
    aggpair = _scatter(m, dsti_s, jnp.zeros((CH, D), f32))
    return _node_mlp(h, aggpair, extra, Wn1, Wn2, bn2, ln_g, ln_b)
